# baseline (speedup 1.0000x reference)
import jax
import jax.numpy as jnp
from jax import lax
from jax.experimental import pallas as pl
from jax.experimental.pallas import tpu as pltpu
from jax.experimental.pallas import tpu_sc as plsc

N_ATOMS = 2097152
N_TYPES = 100
N_SYSTEMS = 8192

NUM_CORES = 2
NUM_SUBCORES = 16
NW = NUM_CORES * NUM_SUBCORES
ATOMS_PER_W = N_ATOMS // NW
CHUNK = 4096
NCHUNKS = ATOMS_PER_W // CHUNK
VECS = CHUNK // 16
UNROLL = 8


def _sc_body(types_hbm, sys_hbm, w_hbm, part_hbm, wbuf, tbuf, sbuf, acc,
             sem_t, sem_s):
  wid = lax.axis_index("s") * NUM_CORES + lax.axis_index("c")
  base = wid * ATOMS_PER_W

  def chunk_refs(c):
    off = (c % 2) * CHUNK
    cbase = base + c * CHUNK
    slot = c % 2
    return (
        (types_hbm.at[pl.ds(cbase, CHUNK)], tbuf.at[pl.ds(off, CHUNK)],
         sem_t.at[slot]),
        (sys_hbm.at[pl.ds(cbase, CHUNK)], sbuf.at[pl.ds(off, CHUNK)],
         sem_s.at[slot]),
    )

  def start_chunk(c):
    for src, dst, sem in chunk_refs(c):
      pltpu.async_copy(src, dst, sem)

  start_chunk(0)
  start_chunk(1)
  pltpu.sync_copy(w_hbm, wbuf)

  @plsc.parallel_loop(0, N_SYSTEMS // 16, 1, unroll=8)
  def zero_body(i):
    acc[pl.ds(i * 16, 16)] = jnp.zeros((16,), jnp.float32)

  def chunk_body(c, carry):
    for src, dst, sem in chunk_refs(c):
      pltpu.make_async_copy(src, dst, sem).wait()
    off = (c % 2) * CHUNK

    @plsc.parallel_loop(0, VECS, 1, unroll=UNROLL)
    def vec_body(k):
      s = sbuf[pl.ds(off + k * 16, 16)]
      cnt, run_end = plsc.scan_count(s)
      run_start = cnt == 1
      t = tbuf[pl.ds(off + k * 16, 16)]
      vw = plsc.load_gather(wbuf, [t])
      cs = plsc.cumsum(vw)
      plsc.addupdate_scatter(acc, [s], cs, mask=run_end)
      plsc.addupdate_scatter(acc, [s], vw - cs, mask=run_start)

    @pl.when(c + 2 < NCHUNKS)
    def _():
      start_chunk(c + 2)

    return carry

  lax.fori_loop(0, NCHUNKS, chunk_body, 0)
  pltpu.sync_copy(acc, part_hbm.at[wid])


def _merge_body(p_ref, o_ref):
  o_ref[...] = jnp.sum(p_ref[...], axis=0, keepdims=True)


@jax.jit
def kernel(types, system_ids, weights):
  sc_fn = pl.kernel(
      _sc_body,
      out_type=jax.ShapeDtypeStruct((NW, N_SYSTEMS), jnp.float32),
      mesh=plsc.VectorSubcoreMesh(core_axis_name="c", subcore_axis_name="s"),
      compiler_params=pltpu.CompilerParams(needs_layout_passes=False),
      scratch_types=[
          pltpu.VMEM((N_TYPES,), jnp.float32),
          pltpu.VMEM((2 * CHUNK,), jnp.int32),
          pltpu.VMEM((2 * CHUNK,), jnp.int32),
          pltpu.VMEM((N_SYSTEMS,), jnp.float32),
          pltpu.SemaphoreType.DMA((2,)),
          pltpu.SemaphoreType.DMA((2,)),
      ],
  )
  partials = sc_fn(types, system_ids, weights)

  merged = pl.pallas_call(
      _merge_body,
      out_shape=jax.ShapeDtypeStruct((1, N_SYSTEMS), jnp.float32),
  )(partials)
  return merged.reshape(N_SYSTEMS, 1)


# speedup vs baseline: 1.0026x; 1.0026x over previous
import jax
import jax.numpy as jnp
from jax import lax
from jax.experimental import pallas as pl
from jax.experimental.pallas import tpu as pltpu
from jax.experimental.pallas import tpu_sc as plsc

N_ATOMS = 2097152
N_TYPES = 100
N_SYSTEMS = 8192

NUM_CORES = 2
NUM_SUBCORES = 16
NW = NUM_CORES * NUM_SUBCORES
ATOMS_PER_W = N_ATOMS // NW
CHUNK = 8192
NCHUNKS = ATOMS_PER_W // CHUNK
VECS = CHUNK // 16
UNROLL = 8


def _sc_body(types_hbm, sys_hbm, w_hbm, part_hbm, wbuf, tbuf, sbuf, acc,
             sem_t, sem_s):
  wid = lax.axis_index("s") * NUM_CORES + lax.axis_index("c")
  base = wid * ATOMS_PER_W

  def chunk_refs(c):
    off = (c % 2) * CHUNK
    cbase = base + c * CHUNK
    slot = c % 2
    return (
        (types_hbm.at[pl.ds(cbase, CHUNK)], tbuf.at[pl.ds(off, CHUNK)],
         sem_t.at[slot]),
        (sys_hbm.at[pl.ds(cbase, CHUNK)], sbuf.at[pl.ds(off, CHUNK)],
         sem_s.at[slot]),
    )

  def start_chunk(c):
    for src, dst, sem in chunk_refs(c):
      pltpu.async_copy(src, dst, sem)

  start_chunk(0)
  start_chunk(1)
  pltpu.sync_copy(w_hbm, wbuf)

  @plsc.parallel_loop(0, N_SYSTEMS // 16, 1, unroll=8)
  def zero_body(i):
    acc[pl.ds(i * 16, 16)] = jnp.zeros((16,), jnp.float32)

  def chunk_body(c, carry):
    for src, dst, sem in chunk_refs(c):
      pltpu.make_async_copy(src, dst, sem).wait()
    off = (c % 2) * CHUNK

    @plsc.parallel_loop(0, VECS, 1, unroll=UNROLL)
    def vec_body(k):
      s = sbuf[pl.ds(off + k * 16, 16)]
      cnt, run_end = plsc.scan_count(s)
      run_start = cnt == 1
      t = tbuf[pl.ds(off + k * 16, 16)]
      vw = plsc.load_gather(wbuf, [t])
      cs = plsc.cumsum(vw)
      plsc.addupdate_scatter(acc, [s], cs, mask=run_end)
      plsc.addupdate_scatter(acc, [s], vw - cs, mask=run_start)

    @pl.when(c + 2 < NCHUNKS)
    def _():
      start_chunk(c + 2)

    return carry

  lax.fori_loop(0, NCHUNKS, chunk_body, 0)
  pltpu.sync_copy(acc, part_hbm.at[wid])


def _merge_body(p_ref, o_ref):
  o_ref[...] = jnp.sum(p_ref[...], axis=0, keepdims=True)


@jax.jit
def kernel(types, system_ids, weights):
  sc_fn = pl.kernel(
      _sc_body,
      out_type=jax.ShapeDtypeStruct((NW, N_SYSTEMS), jnp.float32),
      mesh=plsc.VectorSubcoreMesh(core_axis_name="c", subcore_axis_name="s"),
      compiler_params=pltpu.CompilerParams(needs_layout_passes=False),
      scratch_types=[
          pltpu.VMEM((N_TYPES,), jnp.float32),
          pltpu.VMEM((2 * CHUNK,), jnp.int32),
          pltpu.VMEM((2 * CHUNK,), jnp.int32),
          pltpu.VMEM((N_SYSTEMS,), jnp.float32),
          pltpu.SemaphoreType.DMA((2,)),
          pltpu.SemaphoreType.DMA((2,)),
      ],
  )
  partials = sc_fn(types, system_ids, weights)

  merged = pl.pallas_call(
      _merge_body,
      out_shape=jax.ShapeDtypeStruct((1, N_SYSTEMS), jnp.float32),
  )(partials)
  return merged.reshape(N_SYSTEMS, 1)
